# baseline (speedup 1.0000x reference)
import functools

import jax
import jax.numpy as jnp
from jax import lax
from jax.experimental import pallas as pl
from jax.experimental.pallas import tpu as pltpu
from jax.experimental.pallas import tpu_sc as plsc

NN = 10000
EE = 320000
DD = 128
HH = 4
NEG_SLOPE = 0.2
NCH = 4
CW = DD // NCH
NC = 2
NS = 16
KB = 80
LANES = 16
NPAD = 10112
ROWS_PER_TILE = NPAD // NS
E_PER_W1 = EE // (NC * NS)
E_PER_W2 = EE // NS
NB1 = E_PER_W1 // KB
NB2 = E_PER_W2 // KB

_mesh = plsc.VectorSubcoreMesh(core_axis_name="c", subcore_axis_name="s")


def _p1_body(src_hbm, dst_hbm, ev_hbm, s_hbm, z8_hbm, e_out, den_out,
             stab, srcb, dstb, evb, ebuf, dbuf, accd):
  c = lax.axis_index("c")
  s = lax.axis_index("s")
  wid = c * NS + s
  ebase = wid * E_PER_W1
  pltpu.sync_copy(s_hbm, stab)
  pltpu.sync_copy(z8_hbm, accd.at[pl.ds(s * ROWS_PER_TILE, ROWS_PER_TILE)])
  iota = lax.broadcasted_iota(jnp.int32, (LANES,), 0)

  def zg(g, _):
    rows = g * LANES + iota
    for h in range(HH, 8):
      plsc.store_scatter(dbuf, [rows, jnp.full((LANES,), h, jnp.int32)],
                         jnp.zeros((LANES,), jnp.float32))
    return 0
  lax.fori_loop(0, KB // LANES, zg, 0)
  plsc.subcore_barrier()

  def batch(b, _):
    base = ebase + b * KB
    pltpu.sync_copy(src_hbm.at[pl.ds(base, KB)], srcb)
    pltpu.sync_copy(dst_hbm.at[pl.ds(base, KB)], dstb)
    pltpu.sync_copy(ev_hbm.at[pl.ds(base, KB)], evb)

    def grp(g, _):
      srcv = srcb[pl.ds(g * LANES, LANES)]
      dstv = dstb[pl.ds(g * LANES, LANES)]
      evv = evb[pl.ds(g * LANES, LANES)]
      s8 = srcv * 8
      d8 = dstv * 8 + HH
      rows = g * LANES + iota
      for h in range(HH):
        a1 = plsc.load_gather(stab, [s8 + h])
        a2 = plsc.load_gather(stab, [d8 + h])
        l = a1 + a2
        l = jnp.maximum(l, l * NEG_SLOPE)
        eh = jnp.exp(l) * evv
        ebuf[pl.ds(h * KB + g * LANES, LANES)] = eh
        plsc.store_scatter(dbuf, [rows, jnp.full((LANES,), h, jnp.int32)], eh)
      return 0
    lax.fori_loop(0, KB // LANES, grp, 0)

    gb = wid * NB1 + b
    pltpu.sync_copy(ebuf, e_out.at[pl.ds(gb * (HH * KB), HH * KB)])
    pltpu.sync_copy(dbuf, accd.at[srcb], add=True)
    return 0
  lax.fori_loop(0, NB1, batch, 0)

  plsc.subcore_barrier()
  pltpu.sync_copy(accd.at[pl.ds(s * ROWS_PER_TILE, ROWS_PER_TILE)],
                  den_out.at[pl.ds(c * NPAD + s * ROWS_PER_TILE, ROWS_PER_TILE)])


SB = 10
GD = 4
NSB = NB2 // SB


def _p2_body(src2_hbm, dst2_hbm, e2_hbm, xtab_hbm, z128_hbm, num_out,
             src2d, dstoff2d, eb2d, xb4, pb2, sidx0, sidx1, acc,
             gs0, gs1, gs2, gs3, ss0, ss1):
  sidx2 = [sidx0, sidx1]
  c = lax.axis_index("c")
  s = lax.axis_index("s")
  iota = lax.broadcasted_iota(jnp.int32, (LANES,), 0)
  gsems = [gs0, gs1, gs2, gs3]
  ssems = [ss0, ss1]

  def passbody(p, _):
    chunk = 2 * c + p
    coff = chunk * NN
    pltpu.sync_copy(z128_hbm, acc.at[pl.ds(s * ROWS_PER_TILE, ROWS_PER_TILE)])
    plsc.subcore_barrier()

    def sbody(sb, _):
      row0 = s * NB2 + sb * SB
      pltpu.sync_copy(src2_hbm.at[pl.ds(row0, SB)], src2d)
      pltpu.sync_copy(dst2_hbm.at[pl.ds(row0, SB)], dstoff2d)
      pltpu.sync_copy(e2_hbm.at[pl.ds(row0, SB)], eb2d)

      for t0 in range(SB):
        def offg(k, _, t0=t0):
          v = dstoff2d[t0, pl.ds(k * LANES, LANES)]
          dstoff2d[t0, pl.ds(k * LANES, LANES)] = v + coff
          return 0
        lax.fori_loop(0, KB // LANES, offg, 0)

      desc_s = {}
      desc_g = {}
      desc_g[0] = pltpu.async_copy(xtab_hbm.at[dstoff2d.at[0]], xb4.at[0],
                                   gsems[0])
      for t in range(SB):
        desc_g[t].wait()
        if t + 1 < SB:
          desc_g[t + 1] = pltpu.async_copy(
              xtab_hbm.at[dstoff2d.at[t + 1]], xb4.at[(t + 1) % GD],
              gsems[(t + 1) % GD])
        if t >= 2:
          pass
        i = t % 2
        pbt = pb2.at[i]
        sidx = sidx2[i]
        slotv = jnp.full((LANES,), t % GD, jnp.int32)
        pslotv = jnp.full((LANES,), i, jnp.int32)
        def scp(k, _):
          sidx[pl.ds(k * LANES, LANES)] = src2d[t, pl.ds(k * LANES, LANES)]
          return 0
        lax.fori_loop(0, KB // LANES, scp, 0)

        def grp(g, _):
          rows = g * LANES + iota
          evs = [eb2d[t, pl.ds(h * KB + g * LANES, LANES)] for h in range(HH)]
          for j in range(CW):
            cj = jnp.full((LANES,), j, jnp.int32)
            xcol = plsc.load_gather(xb4, [slotv, rows, cj])
            for h in range(HH):
              plsc.store_scatter(pb2, [pslotv, rows, jnp.full((LANES,), h * CW + j, jnp.int32)],
                                 xcol * evs[h])
          return 0
        lax.fori_loop(0, KB // LANES, grp, 0)

        pltpu.sync_copy(pbt, acc.at[sidx], add=True)
        desc_s[t] = None
      return 0
    lax.fori_loop(0, NSB, sbody, 0)

    plsc.subcore_barrier()
    pltpu.sync_copy(acc.at[pl.ds(s * ROWS_PER_TILE, ROWS_PER_TILE)],
                    num_out.at[pl.ds(chunk * NPAD + s * ROWS_PER_TILE, ROWS_PER_TILE)])
    plsc.subcore_barrier()
    return 0
  lax.fori_loop(0, 2, passbody, 0)


_phase1 = functools.partial(
    pl.kernel, _p1_body,
    out_type=(jax.ShapeDtypeStruct((EE * HH,), jnp.float32),
              jax.ShapeDtypeStruct((NC * NPAD, 8), jnp.float32)),
    mesh=_mesh,
    compiler_params=pltpu.CompilerParams(use_tc_tiling_on_sc=False, needs_layout_passes=False),
    scratch_types=[
        pltpu.VMEM((NN * 8,), jnp.float32),
        pltpu.VMEM((KB,), jnp.int32),
        pltpu.VMEM((KB,), jnp.int32),
        pltpu.VMEM((KB,), jnp.float32),
        pltpu.VMEM((HH * KB,), jnp.float32),
        pltpu.VMEM((KB, 8), jnp.float32),
        pltpu.VMEM_SHARED((NPAD, 8), jnp.float32),
    ])()

_phase2 = functools.partial(
    pl.kernel, _p2_body,
    out_type=jax.ShapeDtypeStruct((NCH * NPAD, DD), jnp.float32),
    mesh=_mesh,
    compiler_params=pltpu.CompilerParams(use_tc_tiling_on_sc=False, needs_layout_passes=False),
    scratch_types=[
        pltpu.VMEM((SB, KB), jnp.int32),
        pltpu.VMEM((SB, KB), jnp.int32),
        pltpu.VMEM((SB, HH * KB), jnp.float32),
        pltpu.VMEM((GD, KB, CW), jnp.float32),
        pltpu.VMEM((2, KB, DD), jnp.float32),
        pltpu.VMEM((KB,), jnp.int32),
        pltpu.VMEM((KB,), jnp.int32),
        pltpu.VMEM_SHARED((NPAD, DD), jnp.float32),
        pltpu.SemaphoreType.DMA,
        pltpu.SemaphoreType.DMA,
        pltpu.SemaphoreType.DMA,
        pltpu.SemaphoreType.DMA,
        pltpu.SemaphoreType.DMA,
        pltpu.SemaphoreType.DMA,
    ])()


def _mm_body(x_ref, b_ref, o_ref):
  o_ref[...] = jnp.dot(x_ref[...], b_ref[...],
                       preferred_element_type=jnp.float32)


def _ep_body(nu_ref, dd_ref, w_ref, o_ref):
  den = dd_ref[0][:, :HH] + dd_ref[1][:, :HH]
  acc = jnp.zeros(o_ref.shape, jnp.float32)
  for h in range(HH):
    t = nu_ref[:, h, :] * w_ref[h, :][None, :] / den[:, h][:, None]
    acc = acc + jnp.where(t > 0, t, jnp.exp(t) - 1.0)
  o_ref[...] = acc * (1.0 / HH)


def kernel(x, edge_index, edge_values, W, a):
  src = edge_index[0]
  dst = edge_index[1]
  B = jnp.concatenate([(W * a[:, :DD]).T, (W * a[:, DD:]).T], axis=1)

  s_nodes = pl.pallas_call(
      _mm_body,
      grid=(10,),
      in_specs=[pl.BlockSpec((NN // 10, DD), lambda i: (i, 0)),
                pl.BlockSpec((DD, 2 * HH), lambda i: (0, 0))],
      out_specs=pl.BlockSpec((NN // 10, 2 * HH), lambda i: (i, 0)),
      out_shape=jax.ShapeDtypeStruct((NN, 2 * HH), jnp.float32),
  )(x, B)

  z8 = jnp.zeros((ROWS_PER_TILE, 8), jnp.float32)
  z128 = jnp.zeros((ROWS_PER_TILE, DD), jnp.float32)
  xtab = x.reshape(NN, NCH, CW).transpose(1, 0, 2).reshape(NCH * NN, CW)

  e_flat, den2 = _phase1(src, dst, edge_values, s_nodes.reshape(-1), z8)
  numer = _phase2(src.reshape(EE // KB, KB), dst.reshape(EE // KB, KB),
                  e_flat.reshape(EE // KB, HH * KB), xtab, z128)

  nu = numer.reshape(NCH, NPAD, HH, CW)[:, :NN].transpose(1, 2, 0, 3).reshape(NN, HH, DD)
  out = pl.pallas_call(
      _ep_body,
      grid=(10,),
      in_specs=[pl.BlockSpec((NN // 10, HH, DD), lambda i: (i, 0, 0)),
                pl.BlockSpec((NC, NN // 10, 8), lambda i: (0, i, 0)),
                pl.BlockSpec((HH, DD), lambda i: (0, 0))],
      out_specs=pl.BlockSpec((NN // 10, DD), lambda i: (i, 0)),
      out_shape=jax.ShapeDtypeStruct((NN, DD), jnp.float32),
  )(nu, den2.reshape(NC, NPAD, 8)[:, :NN], W)
  return out


# speedup vs baseline: 1.0446x; 1.0446x over previous
import functools

import jax
import jax.numpy as jnp
from jax import lax
from jax.experimental import pallas as pl
from jax.experimental.pallas import tpu as pltpu
from jax.experimental.pallas import tpu_sc as plsc

NN = 10000
EE = 320000
DD = 128
HH = 4
NEG_SLOPE = 0.2
NCH = 4
CW = DD // NCH
NC = 2
NS = 16
KB = 80
LANES = 16
NPAD = 10112
ROWS_PER_TILE = NPAD // NS
E_PER_W1 = EE // (NC * NS)
E_PER_W2 = EE // NS
NB1 = E_PER_W1 // KB
NB2 = E_PER_W2 // KB

_mesh = plsc.VectorSubcoreMesh(core_axis_name="c", subcore_axis_name="s")


def _p1_body(src_hbm, dst_hbm, ev_hbm, s_hbm, z8_hbm, e_out, den_out,
             stab, srcb, dstb, evb, ebuf, dbuf, accd):
  c = lax.axis_index("c")
  s = lax.axis_index("s")
  wid = c * NS + s
  ebase = wid * E_PER_W1
  pltpu.sync_copy(s_hbm, stab)
  pltpu.sync_copy(z8_hbm, accd.at[pl.ds(s * ROWS_PER_TILE, ROWS_PER_TILE)])
  iota = lax.broadcasted_iota(jnp.int32, (LANES,), 0)

  def zg(g, _):
    rows = g * LANES + iota
    for h in range(HH, 8):
      plsc.store_scatter(dbuf, [rows, jnp.full((LANES,), h, jnp.int32)],
                         jnp.zeros((LANES,), jnp.float32))
    return 0
  lax.fori_loop(0, KB // LANES, zg, 0)
  plsc.subcore_barrier()

  def batch(b, _):
    base = ebase + b * KB
    pltpu.sync_copy(src_hbm.at[pl.ds(base, KB)], srcb)
    pltpu.sync_copy(dst_hbm.at[pl.ds(base, KB)], dstb)
    pltpu.sync_copy(ev_hbm.at[pl.ds(base, KB)], evb)

    def grp(g, _):
      srcv = srcb[pl.ds(g * LANES, LANES)]
      dstv = dstb[pl.ds(g * LANES, LANES)]
      evv = evb[pl.ds(g * LANES, LANES)]
      s8 = srcv * 8
      d8 = dstv * 8 + HH
      rows = g * LANES + iota
      for h in range(HH):
        a1 = plsc.load_gather(stab, [s8 + h])
        a2 = plsc.load_gather(stab, [d8 + h])
        l = a1 + a2
        l = jnp.maximum(l, l * NEG_SLOPE)
        eh = jnp.exp(l) * evv
        ebuf[pl.ds(h * KB + g * LANES, LANES)] = eh
        plsc.store_scatter(dbuf, [rows, jnp.full((LANES,), h, jnp.int32)], eh)
      return 0
    lax.fori_loop(0, KB // LANES, grp, 0)

    gb = wid * NB1 + b
    pltpu.sync_copy(ebuf, e_out.at[pl.ds(gb * (HH * KB), HH * KB)])
    pltpu.sync_copy(dbuf, accd.at[srcb], add=True)
    return 0
  lax.fori_loop(0, NB1, batch, 0)

  plsc.subcore_barrier()
  pltpu.sync_copy(accd.at[pl.ds(s * ROWS_PER_TILE, ROWS_PER_TILE)],
                  den_out.at[pl.ds(c * NPAD + s * ROWS_PER_TILE, ROWS_PER_TILE)])


SB = 10
GD = 4
NSB = NB2 // SB


def _p2_body(src2_hbm, dst2_hbm, e2_hbm, xtab_hbm, z128_hbm, num_out,
             src2d, dstoff2d, eb2d, xb4, pb2, sidx0, sidx1, acc,
             gs0, gs1, gs2, gs3, ss0, ss1):
  sidx2 = [sidx0, sidx1]
  c = lax.axis_index("c")
  s = lax.axis_index("s")
  iota = lax.broadcasted_iota(jnp.int32, (LANES,), 0)
  gsems = [gs0, gs1, gs2, gs3]
  ssems = [ss0, ss1]

  def passbody(p, _):
    chunk = 2 * c + p
    coff = chunk * NN
    pltpu.sync_copy(z128_hbm, acc.at[pl.ds(s * ROWS_PER_TILE, ROWS_PER_TILE)])
    plsc.subcore_barrier()

    def sbody(sb, _):
      row0 = s * NB2 + sb * SB
      pltpu.sync_copy(src2_hbm.at[pl.ds(row0, SB)], src2d)
      pltpu.sync_copy(dst2_hbm.at[pl.ds(row0, SB)], dstoff2d)
      pltpu.sync_copy(e2_hbm.at[pl.ds(row0, SB)], eb2d)

      for t0 in range(SB):
        def offg(k, _, t0=t0):
          v = dstoff2d[t0, pl.ds(k * LANES, LANES)]
          dstoff2d[t0, pl.ds(k * LANES, LANES)] = v + coff
          return 0
        lax.fori_loop(0, KB // LANES, offg, 0)

      desc_s = {}
      desc_g = {}
      desc_g[0] = pltpu.async_copy(xtab_hbm.at[dstoff2d.at[0]], xb4.at[0],
                                   gsems[0])
      for t in range(SB):
        desc_g[t].wait()
        if t + 1 < SB:
          desc_g[t + 1] = pltpu.async_copy(
              xtab_hbm.at[dstoff2d.at[t + 1]], xb4.at[(t + 1) % GD],
              gsems[(t + 1) % GD])
        if t >= 2:
          desc_s[t - 2].wait()
        i = t % 2
        pbt = pb2.at[i]
        sidx = sidx2[i]
        slotv = jnp.full((LANES,), t % GD, jnp.int32)
        pslotv = jnp.full((LANES,), i, jnp.int32)
        def scp(k, _):
          sidx[pl.ds(k * LANES, LANES)] = src2d[t, pl.ds(k * LANES, LANES)]
          return 0
        lax.fori_loop(0, KB // LANES, scp, 0)

        def grp(g, _):
          rows = g * LANES + iota
          evs = [eb2d[t, pl.ds(h * KB + g * LANES, LANES)] for h in range(HH)]
          for j in range(CW):
            cj = jnp.full((LANES,), j, jnp.int32)
            xcol = plsc.load_gather(xb4, [slotv, rows, cj])
            for h in range(HH):
              plsc.store_scatter(pb2, [pslotv, rows, jnp.full((LANES,), h * CW + j, jnp.int32)],
                                 xcol * evs[h])
          return 0
        lax.fori_loop(0, KB // LANES, grp, 0)

        desc_s[t] = pltpu.async_copy(pbt, acc.at[sidx], ssems[i], add=True)
      desc_s[SB - 2].wait()
      desc_s[SB - 1].wait()
      return 0
    lax.fori_loop(0, NSB, sbody, 0)

    plsc.subcore_barrier()
    pltpu.sync_copy(acc.at[pl.ds(s * ROWS_PER_TILE, ROWS_PER_TILE)],
                    num_out.at[pl.ds(chunk * NPAD + s * ROWS_PER_TILE, ROWS_PER_TILE)])
    plsc.subcore_barrier()
    return 0
  lax.fori_loop(0, 2, passbody, 0)


_phase1 = functools.partial(
    pl.kernel, _p1_body,
    out_type=(jax.ShapeDtypeStruct((EE * HH,), jnp.float32),
              jax.ShapeDtypeStruct((NC * NPAD, 8), jnp.float32)),
    mesh=_mesh,
    compiler_params=pltpu.CompilerParams(use_tc_tiling_on_sc=False, needs_layout_passes=False),
    scratch_types=[
        pltpu.VMEM((NN * 8,), jnp.float32),
        pltpu.VMEM((KB,), jnp.int32),
        pltpu.VMEM((KB,), jnp.int32),
        pltpu.VMEM((KB,), jnp.float32),
        pltpu.VMEM((HH * KB,), jnp.float32),
        pltpu.VMEM((KB, 8), jnp.float32),
        pltpu.VMEM_SHARED((NPAD, 8), jnp.float32),
    ])()

_phase2 = functools.partial(
    pl.kernel, _p2_body,
    out_type=jax.ShapeDtypeStruct((NCH * NPAD, DD), jnp.float32),
    mesh=_mesh,
    compiler_params=pltpu.CompilerParams(use_tc_tiling_on_sc=False, needs_layout_passes=False),
    scratch_types=[
        pltpu.VMEM((SB, KB), jnp.int32),
        pltpu.VMEM((SB, KB), jnp.int32),
        pltpu.VMEM((SB, HH * KB), jnp.float32),
        pltpu.VMEM((GD, KB, CW), jnp.float32),
        pltpu.VMEM((2, KB, DD), jnp.float32),
        pltpu.VMEM((KB,), jnp.int32),
        pltpu.VMEM((KB,), jnp.int32),
        pltpu.VMEM_SHARED((NPAD, DD), jnp.float32),
        pltpu.SemaphoreType.DMA,
        pltpu.SemaphoreType.DMA,
        pltpu.SemaphoreType.DMA,
        pltpu.SemaphoreType.DMA,
        pltpu.SemaphoreType.DMA,
        pltpu.SemaphoreType.DMA,
    ])()


def _mm_body(x_ref, b_ref, o_ref):
  o_ref[...] = jnp.dot(x_ref[...], b_ref[...],
                       preferred_element_type=jnp.float32)


def _ep_body(nu_ref, dd_ref, w_ref, o_ref):
  den = dd_ref[0][:, :HH] + dd_ref[1][:, :HH]
  acc = jnp.zeros(o_ref.shape, jnp.float32)
  for h in range(HH):
    t = nu_ref[:, h, :] * w_ref[h, :][None, :] / den[:, h][:, None]
    acc = acc + jnp.where(t > 0, t, jnp.exp(t) - 1.0)
  o_ref[...] = acc * (1.0 / HH)


def kernel(x, edge_index, edge_values, W, a):
  src = edge_index[0]
  dst = edge_index[1]
  B = jnp.concatenate([(W * a[:, :DD]).T, (W * a[:, DD:]).T], axis=1)

  s_nodes = pl.pallas_call(
      _mm_body,
      grid=(10,),
      in_specs=[pl.BlockSpec((NN // 10, DD), lambda i: (i, 0)),
                pl.BlockSpec((DD, 2 * HH), lambda i: (0, 0))],
      out_specs=pl.BlockSpec((NN // 10, 2 * HH), lambda i: (i, 0)),
      out_shape=jax.ShapeDtypeStruct((NN, 2 * HH), jnp.float32),
  )(x, B)

  z8 = jnp.zeros((ROWS_PER_TILE, 8), jnp.float32)
  z128 = jnp.zeros((ROWS_PER_TILE, DD), jnp.float32)
  xtab = x.reshape(NN, NCH, CW).transpose(1, 0, 2).reshape(NCH * NN, CW)

  e_flat, den2 = _phase1(src, dst, edge_values, s_nodes.reshape(-1), z8)
  numer = _phase2(src.reshape(EE // KB, KB), dst.reshape(EE // KB, KB),
                  e_flat.reshape(EE // KB, HH * KB), xtab, z128)

  nu = numer.reshape(NCH, NPAD, HH, CW)[:, :NN].transpose(1, 2, 0, 3).reshape(NN, HH, DD)
  out = pl.pallas_call(
      _ep_body,
      grid=(10,),
      in_specs=[pl.BlockSpec((NN // 10, HH, DD), lambda i: (i, 0, 0)),
                pl.BlockSpec((NC, NN // 10, 8), lambda i: (0, i, 0)),
                pl.BlockSpec((HH, DD), lambda i: (0, 0))],
      out_specs=pl.BlockSpec((NN // 10, DD), lambda i: (i, 0)),
      out_shape=jax.ShapeDtypeStruct((NN, DD), jnp.float32),
  )(nu, den2.reshape(NC, NPAD, 8)[:, :NN], W)
  return out


# speedup vs baseline: 1.0766x; 1.0306x over previous
import functools

import jax
import jax.numpy as jnp
from jax import lax
from jax.experimental import pallas as pl
from jax.experimental.pallas import tpu as pltpu
from jax.experimental.pallas import tpu_sc as plsc

NN = 10000
EE = 320000
DD = 128
HH = 4
NEG_SLOPE = 0.2
NCH = 4
CW = DD // NCH
NC = 2
NS = 16
KB = 80
LANES = 16
NPAD = 10112
ROWS_PER_TILE = NPAD // NS
E_PER_W1 = EE // (NC * NS)
E_PER_W2 = EE // NS
NB1 = E_PER_W1 // KB
NB2 = E_PER_W2 // KB

_mesh = plsc.VectorSubcoreMesh(core_axis_name="c", subcore_axis_name="s")


SB1 = 5
NSB1 = NB1 // SB1


def _p1_body(src2_hbm, dst2_hbm, ev2_hbm, s_hbm, z8_hbm, e_out, den_out,
             stab, src5, dst5, ev5, ebuf2, dbuf2, sidx20, sidx21, accd,
             es0, es1, dsm0, dsm1):
  c = lax.axis_index("c")
  s = lax.axis_index("s")
  wid = c * NS + s
  esems = [es0, es1]
  dsems = [dsm0, dsm1]
  sidx2 = [sidx20, sidx21]
  pltpu.sync_copy(s_hbm, stab)
  pltpu.sync_copy(z8_hbm, accd.at[pl.ds(s * ROWS_PER_TILE, ROWS_PER_TILE)])
  iota = lax.broadcasted_iota(jnp.int32, (LANES,), 0)

  for i in range(2):
    def zg(g, _, i=i):
      rows = g * LANES + iota
      for h in range(HH, 8):
        plsc.store_scatter(dbuf2, [jnp.full((LANES,), i, jnp.int32), rows,
                                   jnp.full((LANES,), h, jnp.int32)],
                           jnp.zeros((LANES,), jnp.float32))
      return 0
    lax.fori_loop(0, KB // LANES, zg, 0)
  plsc.subcore_barrier()

  def sbody(sb, _):
    row0 = wid * NB1 + sb * SB1
    pltpu.sync_copy(src2_hbm.at[pl.ds(row0, SB1)], src5)
    pltpu.sync_copy(dst2_hbm.at[pl.ds(row0, SB1)], dst5)
    pltpu.sync_copy(ev2_hbm.at[pl.ds(row0, SB1)], ev5)

    desc_e = {}
    desc_d = {}
    for t in range(SB1):
      i = t % 2
      if t >= 2:
        desc_e[t - 2].wait()
        desc_d[t - 2].wait()
      iv = jnp.full((LANES,), i, jnp.int32)
      sidx = sidx2[i]

      def grp(g, _, t=t, iv=iv, sidx=sidx, i=i):
        srcv = src5[t, pl.ds(g * LANES, LANES)]
        dstv = dst5[t, pl.ds(g * LANES, LANES)]
        evv = ev5[t, pl.ds(g * LANES, LANES)]
        sidx[pl.ds(g * LANES, LANES)] = srcv
        s8 = srcv * 8
        d8 = dstv * 8 + HH
        rows = g * LANES + iota
        for h in range(HH):
          a1 = plsc.load_gather(stab, [s8 + h])
          a2 = plsc.load_gather(stab, [d8 + h])
          l = a1 + a2
          l = jnp.maximum(l, l * NEG_SLOPE)
          eh = jnp.exp(l) * evv
          ebuf2[i, pl.ds(h * KB + g * LANES, LANES)] = eh
          plsc.store_scatter(dbuf2, [iv, rows, jnp.full((LANES,), h, jnp.int32)], eh)
        return 0
      lax.fori_loop(0, KB // LANES, grp, 0)

      gb = row0 + t
      desc_e[t] = pltpu.async_copy(
          ebuf2.at[i], e_out.at[pl.ds(gb * (HH * KB), HH * KB)], esems[i])
      desc_d[t] = pltpu.async_copy(dbuf2.at[i], accd.at[sidx], dsems[i],
                                   add=True)
    desc_e[SB1 - 2].wait()
    desc_e[SB1 - 1].wait()
    desc_d[SB1 - 2].wait()
    desc_d[SB1 - 1].wait()
    return 0
  lax.fori_loop(0, NSB1, sbody, 0)

  plsc.subcore_barrier()
  pltpu.sync_copy(accd.at[pl.ds(s * ROWS_PER_TILE, ROWS_PER_TILE)],
                  den_out.at[pl.ds(c * NPAD + s * ROWS_PER_TILE, ROWS_PER_TILE)])


SB = 10
GD = 4
NSB = NB2 // SB


def _p2_body(src2_hbm, dst2_hbm, e2_hbm, xtab_hbm, z128_hbm, num_out,
             src2d, dstoff2d, eb2d, xb4, pb2, sidx0, sidx1, acc,
             gs0, gs1, gs2, gs3, ss0, ss1):
  sidx2 = [sidx0, sidx1]
  c = lax.axis_index("c")
  s = lax.axis_index("s")
  iota = lax.broadcasted_iota(jnp.int32, (LANES,), 0)
  gsems = [gs0, gs1, gs2, gs3]
  ssems = [ss0, ss1]

  def passbody(p, _):
    chunk = 2 * c + p
    coff = chunk * NN
    pltpu.sync_copy(z128_hbm, acc.at[pl.ds(s * ROWS_PER_TILE, ROWS_PER_TILE)])
    plsc.subcore_barrier()

    def sbody(sb, _):
      row0 = s * NB2 + sb * SB
      pltpu.sync_copy(src2_hbm.at[pl.ds(row0, SB)], src2d)
      pltpu.sync_copy(dst2_hbm.at[pl.ds(row0, SB)], dstoff2d)
      pltpu.sync_copy(e2_hbm.at[pl.ds(row0, SB)], eb2d)

      for t0 in range(SB):
        def offg(k, _, t0=t0):
          v = dstoff2d[t0, pl.ds(k * LANES, LANES)]
          dstoff2d[t0, pl.ds(k * LANES, LANES)] = v + coff
          return 0
        lax.fori_loop(0, KB // LANES, offg, 0)

      desc_s = {}
      desc_g = {}
      desc_g[0] = pltpu.async_copy(xtab_hbm.at[dstoff2d.at[0]], xb4.at[0],
                                   gsems[0])
      for t in range(SB):
        desc_g[t].wait()
        if t + 1 < SB:
          desc_g[t + 1] = pltpu.async_copy(
              xtab_hbm.at[dstoff2d.at[t + 1]], xb4.at[(t + 1) % GD],
              gsems[(t + 1) % GD])
        if t >= 2:
          desc_s[t - 2].wait()
        i = t % 2
        pbt = pb2.at[i]
        sidx = sidx2[i]
        slotv = jnp.full((LANES,), t % GD, jnp.int32)
        pslotv = jnp.full((LANES,), i, jnp.int32)
        def scp(k, _):
          sidx[pl.ds(k * LANES, LANES)] = src2d[t, pl.ds(k * LANES, LANES)]
          return 0
        lax.fori_loop(0, KB // LANES, scp, 0)

        def grp(g, _):
          rows = g * LANES + iota
          evs = [eb2d[t, pl.ds(h * KB + g * LANES, LANES)] for h in range(HH)]
          for j in range(CW):
            cj = jnp.full((LANES,), j, jnp.int32)
            xcol = plsc.load_gather(xb4, [slotv, rows, cj])
            for h in range(HH):
              plsc.store_scatter(pb2, [pslotv, rows, jnp.full((LANES,), h * CW + j, jnp.int32)],
                                 xcol * evs[h])
          return 0
        lax.fori_loop(0, KB // LANES, grp, 0)

        desc_s[t] = pltpu.async_copy(pbt, acc.at[sidx], ssems[i], add=True)
      desc_s[SB - 2].wait()
      desc_s[SB - 1].wait()
      return 0
    lax.fori_loop(0, NSB, sbody, 0)

    plsc.subcore_barrier()
    pltpu.sync_copy(acc.at[pl.ds(s * ROWS_PER_TILE, ROWS_PER_TILE)],
                    num_out.at[pl.ds(chunk * NPAD + s * ROWS_PER_TILE, ROWS_PER_TILE)])
    plsc.subcore_barrier()
    return 0
  lax.fori_loop(0, 2, passbody, 0)


_phase1 = functools.partial(
    pl.kernel, _p1_body,
    out_type=(jax.ShapeDtypeStruct((EE * HH,), jnp.float32),
              jax.ShapeDtypeStruct((NC * NPAD, 8), jnp.float32)),
    mesh=_mesh,
    compiler_params=pltpu.CompilerParams(use_tc_tiling_on_sc=False, needs_layout_passes=False),
    scratch_types=[
        pltpu.VMEM((NN * 8,), jnp.float32),
        pltpu.VMEM((SB1, KB), jnp.int32),
        pltpu.VMEM((SB1, KB), jnp.int32),
        pltpu.VMEM((SB1, KB), jnp.float32),
        pltpu.VMEM((2, HH * KB), jnp.float32),
        pltpu.VMEM((2, KB, 8), jnp.float32),
        pltpu.VMEM((KB,), jnp.int32),
        pltpu.VMEM((KB,), jnp.int32),
        pltpu.VMEM_SHARED((NPAD, 8), jnp.float32),
        pltpu.SemaphoreType.DMA,
        pltpu.SemaphoreType.DMA,
        pltpu.SemaphoreType.DMA,
        pltpu.SemaphoreType.DMA,
    ])()

_phase2 = functools.partial(
    pl.kernel, _p2_body,
    out_type=jax.ShapeDtypeStruct((NCH * NPAD, DD), jnp.float32),
    mesh=_mesh,
    compiler_params=pltpu.CompilerParams(use_tc_tiling_on_sc=False, needs_layout_passes=False),
    scratch_types=[
        pltpu.VMEM((SB, KB), jnp.int32),
        pltpu.VMEM((SB, KB), jnp.int32),
        pltpu.VMEM((SB, HH * KB), jnp.float32),
        pltpu.VMEM((GD, KB, CW), jnp.float32),
        pltpu.VMEM((2, KB, DD), jnp.float32),
        pltpu.VMEM((KB,), jnp.int32),
        pltpu.VMEM((KB,), jnp.int32),
        pltpu.VMEM_SHARED((NPAD, DD), jnp.float32),
        pltpu.SemaphoreType.DMA,
        pltpu.SemaphoreType.DMA,
        pltpu.SemaphoreType.DMA,
        pltpu.SemaphoreType.DMA,
        pltpu.SemaphoreType.DMA,
        pltpu.SemaphoreType.DMA,
    ])()


def _mm_body(x_ref, b_ref, o_ref):
  o_ref[...] = jnp.dot(x_ref[...], b_ref[...],
                       preferred_element_type=jnp.float32)


def _ep_body(nu_ref, dd_ref, w_ref, o_ref):
  den = dd_ref[0][:, :HH] + dd_ref[1][:, :HH]
  acc = jnp.zeros(o_ref.shape, jnp.float32)
  for h in range(HH):
    t = nu_ref[:, h, :] * w_ref[h, :][None, :] / den[:, h][:, None]
    acc = acc + jnp.where(t > 0, t, jnp.exp(t) - 1.0)
  o_ref[...] = acc * (1.0 / HH)


def kernel(x, edge_index, edge_values, W, a):
  src = edge_index[0]
  dst = edge_index[1]
  B = jnp.concatenate([(W * a[:, :DD]).T, (W * a[:, DD:]).T], axis=1)

  s_nodes = pl.pallas_call(
      _mm_body,
      grid=(10,),
      in_specs=[pl.BlockSpec((NN // 10, DD), lambda i: (i, 0)),
                pl.BlockSpec((DD, 2 * HH), lambda i: (0, 0))],
      out_specs=pl.BlockSpec((NN // 10, 2 * HH), lambda i: (i, 0)),
      out_shape=jax.ShapeDtypeStruct((NN, 2 * HH), jnp.float32),
  )(x, B)

  z8 = jnp.zeros((ROWS_PER_TILE, 8), jnp.float32)
  z128 = jnp.zeros((ROWS_PER_TILE, DD), jnp.float32)
  xtab = x.reshape(NN, NCH, CW).transpose(1, 0, 2).reshape(NCH * NN, CW)

  e_flat, den2 = _phase1(src.reshape(EE // KB, KB), dst.reshape(EE // KB, KB),
                         edge_values.reshape(EE // KB, KB),
                         s_nodes.reshape(-1), z8)
  numer = _phase2(src.reshape(EE // KB, KB), dst.reshape(EE // KB, KB),
                  e_flat.reshape(EE // KB, HH * KB), xtab, z128)

  nu = numer.reshape(NCH, NPAD, HH, CW)[:, :NN].transpose(1, 2, 0, 3).reshape(NN, HH, DD)
  out = pl.pallas_call(
      _ep_body,
      grid=(10,),
      in_specs=[pl.BlockSpec((NN // 10, HH, DD), lambda i: (i, 0, 0)),
                pl.BlockSpec((NC, NN // 10, 8), lambda i: (0, i, 0)),
                pl.BlockSpec((HH, DD), lambda i: (0, 0))],
      out_specs=pl.BlockSpec((NN // 10, DD), lambda i: (i, 0)),
      out_shape=jax.ShapeDtypeStruct((NN, DD), jnp.float32),
  )(nu, den2.reshape(NC, NPAD, 8)[:, :NN], W)
  return out


# speedup vs baseline: 1.0774x; 1.0008x over previous
import functools

import jax
import jax.numpy as jnp
from jax import lax
from jax.experimental import pallas as pl
from jax.experimental.pallas import tpu as pltpu
from jax.experimental.pallas import tpu_sc as plsc

NN = 10000
EE = 320000
DD = 128
HH = 4
NEG_SLOPE = 0.2
NCH = 4
CW = DD // NCH
NC = 2
NS = 16
KB = 80
LANES = 16
NPAD = 10112
ROWS_PER_TILE = NPAD // NS
E_PER_W1 = EE // (NC * NS)
E_PER_W2 = EE // NS
NB1 = E_PER_W1 // KB
NB2 = E_PER_W2 // KB

_mesh = plsc.VectorSubcoreMesh(core_axis_name="c", subcore_axis_name="s")


SB1 = 5
NSB1 = NB1 // SB1


def _p1_body(src2_hbm, dst2_hbm, ev2_hbm, s_hbm, z8_hbm, e_out, den_out,
             stab, src5, dst5, ev5, ebuf2, dbuf2, sidx20, sidx21, accd,
             es0, es1, dsm0, dsm1):
  c = lax.axis_index("c")
  s = lax.axis_index("s")
  wid = c * NS + s
  esems = [es0, es1]
  dsems = [dsm0, dsm1]
  sidx2 = [sidx20, sidx21]
  pltpu.sync_copy(s_hbm, stab)
  pltpu.sync_copy(z8_hbm, accd.at[pl.ds(s * ROWS_PER_TILE, ROWS_PER_TILE)])
  iota = lax.broadcasted_iota(jnp.int32, (LANES,), 0)

  for i in range(2):
    def zg(g, _, i=i):
      rows = g * LANES + iota
      for h in range(HH, 8):
        plsc.store_scatter(dbuf2, [jnp.full((LANES,), i, jnp.int32), rows,
                                   jnp.full((LANES,), h, jnp.int32)],
                           jnp.zeros((LANES,), jnp.float32))
      return 0
    lax.fori_loop(0, KB // LANES, zg, 0)
  plsc.subcore_barrier()

  def sbody(sb, _):
    row0 = wid * NB1 + sb * SB1
    pltpu.sync_copy(src2_hbm.at[pl.ds(row0, SB1)], src5)
    pltpu.sync_copy(dst2_hbm.at[pl.ds(row0, SB1)], dst5)
    pltpu.sync_copy(ev2_hbm.at[pl.ds(row0, SB1)], ev5)

    desc_e = {}
    desc_d = {}
    for t in range(SB1):
      i = t % 2
      if t >= 2:
        desc_e[t - 2].wait()
        desc_d[t - 2].wait()
      iv = jnp.full((LANES,), i, jnp.int32)
      sidx = sidx2[i]

      def grp(g, _, t=t, iv=iv, sidx=sidx, i=i):
        srcv = src5[t, pl.ds(g * LANES, LANES)]
        dstv = dst5[t, pl.ds(g * LANES, LANES)]
        evv = ev5[t, pl.ds(g * LANES, LANES)]
        sidx[pl.ds(g * LANES, LANES)] = srcv
        s8 = srcv * 8
        d8 = dstv * 8 + HH
        rows = g * LANES + iota
        for h in range(HH):
          a1 = plsc.load_gather(stab, [s8 + h])
          a2 = plsc.load_gather(stab, [d8 + h])
          l = a1 + a2
          l = jnp.maximum(l, l * NEG_SLOPE)
          eh = jnp.exp(l) * evv
          ebuf2[i, pl.ds(h * KB + g * LANES, LANES)] = eh
          plsc.store_scatter(dbuf2, [iv, rows, jnp.full((LANES,), h, jnp.int32)], eh)
        return 0
      lax.fori_loop(0, KB // LANES, grp, 0)

      gb = row0 + t
      desc_e[t] = pltpu.async_copy(
          ebuf2.at[i], e_out.at[pl.ds(gb * (HH * KB), HH * KB)], esems[i])
      desc_d[t] = pltpu.async_copy(dbuf2.at[i], accd.at[sidx], dsems[i],
                                   add=True)
    desc_e[SB1 - 2].wait()
    desc_e[SB1 - 1].wait()
    desc_d[SB1 - 2].wait()
    desc_d[SB1 - 1].wait()
    return 0
  lax.fori_loop(0, NSB1, sbody, 0)

  plsc.subcore_barrier()
  pltpu.sync_copy(accd.at[pl.ds(s * ROWS_PER_TILE, ROWS_PER_TILE)],
                  den_out.at[pl.ds(c * NPAD + s * ROWS_PER_TILE, ROWS_PER_TILE)])


SB = 10
GD = 4
NSB = NB2 // SB


def _p2_body(src2_hbm, dst2_hbm, e2_hbm, xtab_hbm, z128_hbm, num_out,
             src2d, dstoff2d, eb2d, xb4, pb2, sidx0, sidx1, sidx2b, acc,
             gs0, gs1, gs2, gs3, ss0, ss1, ss2):
  sidx2 = [sidx0, sidx1, sidx2b]
  c = lax.axis_index("c")
  s = lax.axis_index("s")
  iota = lax.broadcasted_iota(jnp.int32, (LANES,), 0)
  gsems = [gs0, gs1, gs2, gs3]
  ssems = [ss0, ss1, ss2]

  def passbody(p, _):
    chunk = 2 * c + p
    coff = chunk * NN
    pltpu.sync_copy(z128_hbm, acc.at[pl.ds(s * ROWS_PER_TILE, ROWS_PER_TILE)])
    plsc.subcore_barrier()

    def sbody(sb, _):
      row0 = s * NB2 + sb * SB
      pltpu.sync_copy(src2_hbm.at[pl.ds(row0, SB)], src2d)
      pltpu.sync_copy(dst2_hbm.at[pl.ds(row0, SB)], dstoff2d)
      pltpu.sync_copy(e2_hbm.at[pl.ds(row0, SB)], eb2d)

      for t0 in range(SB):
        def offg(k, _, t0=t0):
          v = dstoff2d[t0, pl.ds(k * LANES, LANES)]
          dstoff2d[t0, pl.ds(k * LANES, LANES)] = v + coff
          return 0
        lax.fori_loop(0, KB // LANES, offg, 0)

      desc_s = {}
      desc_g = {}
      desc_g[0] = pltpu.async_copy(xtab_hbm.at[dstoff2d.at[0]], xb4.at[0],
                                   gsems[0])
      for t in range(SB):
        desc_g[t].wait()
        if t + 1 < SB:
          desc_g[t + 1] = pltpu.async_copy(
              xtab_hbm.at[dstoff2d.at[t + 1]], xb4.at[(t + 1) % GD],
              gsems[(t + 1) % GD])
        if t >= 3:
          desc_s[t - 3].wait()
        i = t % 3
        pbt = pb2.at[i]
        sidx = sidx2[i]
        slotv = jnp.full((LANES,), t % GD, jnp.int32)
        pslotv = jnp.full((LANES,), i, jnp.int32)
        def scp(k, _):
          sidx[pl.ds(k * LANES, LANES)] = src2d[t, pl.ds(k * LANES, LANES)]
          return 0
        lax.fori_loop(0, KB // LANES, scp, 0)

        def grp(g, _):
          rows = g * LANES + iota
          evs = [eb2d[t, pl.ds(h * KB + g * LANES, LANES)] for h in range(HH)]
          for j in range(CW):
            cj = jnp.full((LANES,), j, jnp.int32)
            xcol = plsc.load_gather(xb4, [slotv, rows, cj])
            for h in range(HH):
              plsc.store_scatter(pb2, [pslotv, rows, jnp.full((LANES,), h * CW + j, jnp.int32)],
                                 xcol * evs[h])
          return 0
        lax.fori_loop(0, KB // LANES, grp, 0)

        desc_s[t] = pltpu.async_copy(pbt, acc.at[sidx], ssems[i], add=True)
      desc_s[SB - 3].wait()
      desc_s[SB - 2].wait()
      desc_s[SB - 1].wait()
      return 0
    lax.fori_loop(0, NSB, sbody, 0)

    plsc.subcore_barrier()
    pltpu.sync_copy(acc.at[pl.ds(s * ROWS_PER_TILE, ROWS_PER_TILE)],
                    num_out.at[pl.ds(chunk * NPAD + s * ROWS_PER_TILE, ROWS_PER_TILE)])
    plsc.subcore_barrier()
    return 0
  lax.fori_loop(0, 2, passbody, 0)


_phase1 = functools.partial(
    pl.kernel, _p1_body,
    out_type=(jax.ShapeDtypeStruct((EE * HH,), jnp.float32),
              jax.ShapeDtypeStruct((NC * NPAD, 8), jnp.float32)),
    mesh=_mesh,
    compiler_params=pltpu.CompilerParams(use_tc_tiling_on_sc=False, needs_layout_passes=False),
    scratch_types=[
        pltpu.VMEM((NN * 8,), jnp.float32),
        pltpu.VMEM((SB1, KB), jnp.int32),
        pltpu.VMEM((SB1, KB), jnp.int32),
        pltpu.VMEM((SB1, KB), jnp.float32),
        pltpu.VMEM((2, HH * KB), jnp.float32),
        pltpu.VMEM((2, KB, 8), jnp.float32),
        pltpu.VMEM((KB,), jnp.int32),
        pltpu.VMEM((KB,), jnp.int32),
        pltpu.VMEM_SHARED((NPAD, 8), jnp.float32),
        pltpu.SemaphoreType.DMA,
        pltpu.SemaphoreType.DMA,
        pltpu.SemaphoreType.DMA,
        pltpu.SemaphoreType.DMA,
    ])()

_phase2 = functools.partial(
    pl.kernel, _p2_body,
    out_type=jax.ShapeDtypeStruct((NCH * NPAD, DD), jnp.float32),
    mesh=_mesh,
    compiler_params=pltpu.CompilerParams(use_tc_tiling_on_sc=False, needs_layout_passes=False),
    scratch_types=[
        pltpu.VMEM((SB, KB), jnp.int32),
        pltpu.VMEM((SB, KB), jnp.int32),
        pltpu.VMEM((SB, HH * KB), jnp.float32),
        pltpu.VMEM((GD, KB, CW), jnp.float32),
        pltpu.VMEM((3, KB, DD), jnp.float32),
        pltpu.VMEM((KB,), jnp.int32),
        pltpu.VMEM((KB,), jnp.int32),
        pltpu.VMEM((KB,), jnp.int32),
        pltpu.VMEM_SHARED((NPAD, DD), jnp.float32),
        pltpu.SemaphoreType.DMA,
        pltpu.SemaphoreType.DMA,
        pltpu.SemaphoreType.DMA,
        pltpu.SemaphoreType.DMA,
        pltpu.SemaphoreType.DMA,
        pltpu.SemaphoreType.DMA,
        pltpu.SemaphoreType.DMA,
    ])()


def _mm_body(x_ref, b_ref, o_ref):
  o_ref[...] = jnp.dot(x_ref[...], b_ref[...],
                       preferred_element_type=jnp.float32)


def _ep_body(nu_ref, dd_ref, w_ref, o_ref):
  den = dd_ref[0][:, :HH] + dd_ref[1][:, :HH]
  acc = jnp.zeros(o_ref.shape, jnp.float32)
  for h in range(HH):
    t = nu_ref[:, h, :] * w_ref[h, :][None, :] / den[:, h][:, None]
    acc = acc + jnp.where(t > 0, t, jnp.exp(t) - 1.0)
  o_ref[...] = acc * (1.0 / HH)


def kernel(x, edge_index, edge_values, W, a):
  src = edge_index[0]
  dst = edge_index[1]
  B = jnp.concatenate([(W * a[:, :DD]).T, (W * a[:, DD:]).T], axis=1)

  s_nodes = pl.pallas_call(
      _mm_body,
      grid=(10,),
      in_specs=[pl.BlockSpec((NN // 10, DD), lambda i: (i, 0)),
                pl.BlockSpec((DD, 2 * HH), lambda i: (0, 0))],
      out_specs=pl.BlockSpec((NN // 10, 2 * HH), lambda i: (i, 0)),
      out_shape=jax.ShapeDtypeStruct((NN, 2 * HH), jnp.float32),
  )(x, B)

  z8 = jnp.zeros((ROWS_PER_TILE, 8), jnp.float32)
  z128 = jnp.zeros((ROWS_PER_TILE, DD), jnp.float32)
  xtab = x.reshape(NN, NCH, CW).transpose(1, 0, 2).reshape(NCH * NN, CW)

  e_flat, den2 = _phase1(src.reshape(EE // KB, KB), dst.reshape(EE // KB, KB),
                         edge_values.reshape(EE // KB, KB),
                         s_nodes.reshape(-1), z8)
  numer = _phase2(src.reshape(EE // KB, KB), dst.reshape(EE // KB, KB),
                  e_flat.reshape(EE // KB, HH * KB), xtab, z128)

  nu = numer.reshape(NCH, NPAD, HH, CW)[:, :NN].transpose(1, 2, 0, 3).reshape(NN, HH, DD)
  out = pl.pallas_call(
      _ep_body,
      grid=(10,),
      in_specs=[pl.BlockSpec((NN // 10, HH, DD), lambda i: (i, 0, 0)),
                pl.BlockSpec((NC, NN // 10, 8), lambda i: (0, i, 0)),
                pl.BlockSpec((HH, DD), lambda i: (0, 0))],
      out_specs=pl.BlockSpec((NN // 10, DD), lambda i: (i, 0)),
      out_shape=jax.ShapeDtypeStruct((NN, DD), jnp.float32),
  )(nu, den2.reshape(NC, NPAD, 8)[:, :NN], W)
  return out
